# baseline (speedup 1.0000x reference)
import functools

import jax
import jax.numpy as jnp
from jax import lax
from jax.experimental import pallas as pl
from jax.experimental.pallas import tpu as pltpu
from jax.experimental.pallas import tpu_sc as plsc

B = 16384
U = 1000000
I = 1000000
H = 16
RPW = 128 // H
NC = 2
NS = 16
L = 16
NW = NC * NS
BPW = B // NW
CH = 128
NCH = BPW // CH
NBC = CH // L

_mesh = plsc.VectorSubcoreMesh(core_axis_name="c", subcore_axis_name="s",
                               num_cores=NC, num_subcores=NS)


@functools.partial(
    pl.kernel,
    out_type=(
        jax.ShapeDtypeStruct((B,), jnp.float32),
        jax.ShapeDtypeStruct((NW, L), jnp.float32),
    ),
    mesh=_mesh,
    compiler_params=pltpu.CompilerParams(needs_layout_passes=False,
                                         use_tc_tiling_on_sc=True),
    scratch_types=[
        pltpu.VMEM((NCH, CH), jnp.int32),
        pltpu.VMEM((NCH, CH), jnp.int32),
        pltpu.VMEM((NCH, CH), jnp.int32),
        pltpu.VMEM((NCH, CH), jnp.int32),
        pltpu.VMEM((BPW,), jnp.int32),
        pltpu.VMEM((BPW,), jnp.int32),
        pltpu.VMEM((CH, 128), jnp.float32),
        pltpu.VMEM((CH, 128), jnp.float32),
        pltpu.VMEM((CH, 128), jnp.float32),
        pltpu.VMEM((CH, 128), jnp.float32),
        pltpu.VMEM((BPW,), jnp.float32),
        pltpu.VMEM((BPW,), jnp.float32),
        pltpu.VMEM((BPW,), jnp.float32),
        pltpu.VMEM((BPW,), jnp.float32),
        pltpu.VMEM((L,), jnp.float32),
        pltpu.VMEM((L,), jnp.float32),
        pltpu.SemaphoreType.DMA,
        pltpu.SemaphoreType.DMA,
        pltpu.SemaphoreType.DMA,
    ],
)
def _mf_sc_kernel(uhi_h, ihi_h, uor_h, ior_h, uoff_h, ioff_h, rating_h,
                  uw_h, iw_h, ub_h, ib_h, bias_h,
                  tgt_h, part_h,
                  uhi_v, ihi_v, uor_v, ior_v, uoff_v, ioff_v,
                  ubuf0, ubuf1, ibuf0, ibuf1,
                  ubr_v, ibr_v, rat_v, out_v, sqa_v, bias_v,
                  semA, semB0, semB1):
    wid = lax.axis_index("s") * NC + lax.axis_index("c")
    base = wid * BPW

    pltpu.sync_copy(uhi_h.at[pl.ds(wid * NCH, NCH)], uhi_v)
    pltpu.sync_copy(ihi_h.at[pl.ds(wid * NCH, NCH)], ihi_v)
    pltpu.sync_copy(uor_h.at[pl.ds(wid * NCH, NCH)], uor_v)
    pltpu.sync_copy(ior_h.at[pl.ds(wid * NCH, NCH)], ior_v)
    pltpu.sync_copy(uoff_h.at[pl.ds(base, BPW)], uoff_v)
    pltpu.sync_copy(ioff_h.at[pl.ds(base, BPW)], ioff_v)
    pltpu.sync_copy(rating_h.at[pl.ds(base, BPW)], rat_v)
    pltpu.sync_copy(bias_h, bias_v)

    bias_copies = []
    for c in range(NCH):
        sl = pl.ds(c * CH, CH)
        bias_copies.append(pltpu.async_copy(ub_h.at[uor_v.at[c]], ubr_v.at[sl], semA))
        bias_copies.append(pltpu.async_copy(ib_h.at[ior_v.at[c]], ibr_v.at[sl], semA))

    ubufs = (ubuf0, ubuf1)
    ibufs = (ibuf0, ibuf1)
    sems = (semB0, semB1)

    def fire(c):
        p = c % 2
        return (pltpu.async_copy(uw_h.at[uhi_v.at[c]], ubufs[p], sems[p]),
                pltpu.async_copy(iw_h.at[ihi_v.at[c]], ibufs[p], sems[p]))

    pending = fire(0)
    for cp in bias_copies:
        cp.wait()

    gbias = bias_v[...]
    lanes = lax.iota(jnp.int32, L)
    sqacc = jnp.zeros((L,), jnp.float32)

    for c in range(NCH):
        p = c % 2
        for cp in pending:
            cp.wait()
        if c + 1 < NCH:
            pending = fire(c + 1)
        ubuf, ibuf = ubufs[p], ibufs[p]
        for j in range(NBC):
            o = c * CH + j * L
            rows = j * L + lanes
            ucol0 = uoff_v[pl.ds(o, L)]
            icol0 = ioff_v[pl.ds(o, L)]
            ub = ubr_v[pl.ds(o, L)]
            ib = ibr_v[pl.ds(o, L)]
            acc = jnp.zeros((L,), jnp.float32)
            for h in range(H):
                gu = plsc.load_gather(ubuf, [rows, ucol0 + h])
                gi = plsc.load_gather(ibuf, [rows, icol0 + h])
                acc = acc + (gu + ub) * (gi + ib)
            out = acc + gbias
            out_v[pl.ds(o, L)] = out
            err = out - rat_v[pl.ds(o, L)]
            sqacc = sqacc + err * err

    sqa_v[...] = sqacc
    pltpu.sync_copy(sqa_v, part_h.at[wid])
    pltpu.sync_copy(out_v, tgt_h.at[pl.ds(base, BPW)])


def kernel(user, item, rating, user_weight, item_weight, user_bias, item_bias,
           bias):
    user = user.astype(jnp.int32)
    item = item.astype(jnp.int32)
    uhi = (user >> 3).reshape(NW * NCH, CH)
    ihi = (item >> 3).reshape(NW * NCH, CH)
    uoff = (user & 7) << 4
    ioff = (item & 7) << 4
    uor = user.reshape(NW * NCH, CH)
    ior = item.reshape(NW * NCH, CH)
    one = jnp.float32(1.0) + rating[0] * jnp.float32(0.0)
    uw128 = user_weight.reshape(U // RPW, 128) * one
    iw128 = item_weight.reshape(I // RPW, 128) * one
    bias16 = jnp.broadcast_to(bias.astype(jnp.float32), (L,))
    target, parts = _mf_sc_kernel(uhi, ihi, uor, ior, uoff, ioff, rating,
                                  uw128, iw128, user_bias.reshape(U),
                                  item_bias.reshape(I), bias16)
    loss = jnp.sum(parts) / B
    return (target, loss)


# speedup vs baseline: 2.8766x; 2.8766x over previous
import functools

import jax
import jax.numpy as jnp
from jax import lax
from jax.experimental import pallas as pl
from jax.experimental.pallas import tpu as pltpu
from jax.experimental.pallas import tpu_sc as plsc

B = 16384
U = 1000000
I = 1000000
H = 16
NC = 2
NS = 16
L = 16
NW = NC * NS
BPW = B // NW
NB = BPW // L
CH = 128
NIDX = BPW * H

NG = (U + 127) // 128
GPW = NG // NW
GC = 2
CPW = GPW // GC
CU = GC * 128
NTAIL = NG - GPW * NW

_mesh = plsc.VectorSubcoreMesh(core_axis_name="c", subcore_axis_name="s",
                               num_cores=NC, num_subcores=NS)
_params = pltpu.CompilerParams(needs_layout_passes=False,
                               use_tc_tiling_on_sc=True)


@functools.partial(
    pl.kernel,
    out_type=(
        jax.ShapeDtypeStruct((H * U,), jnp.float32),
        jax.ShapeDtypeStruct((H * I,), jnp.float32),
    ),
    mesh=_mesh,
    compiler_params=_params,
    scratch_types=[
        pltpu.VMEM((8, CU), jnp.float32),
        pltpu.VMEM((8, CU), jnp.float32),
        pltpu.VMEM((8, CU), jnp.float32),
        pltpu.VMEM((8, CU), jnp.float32),
        pltpu.VMEM((8, CU), jnp.float32),
        pltpu.VMEM((8, CU), jnp.float32),
        pltpu.VMEM((8, CU), jnp.float32),
        pltpu.VMEM((8, CU), jnp.float32),
        pltpu.VMEM((CU * H,), jnp.float32),
        pltpu.VMEM((CU * H,), jnp.float32),
        pltpu.VMEM((CU * H,), jnp.float32),
        pltpu.VMEM((CU * H,), jnp.float32),
        pltpu.SemaphoreType.DMA,
        pltpu.SemaphoreType.DMA,
    ],
)
def _relayout_kernel(uwT_h, iwT_h, uscr_h, iscr_h,
                     uin00, uin01, uin10, uin11,
                     iin00, iin01, iin10, iin11,
                     uout0, uout1, iout0, iout1, semR, semW):
    wid = lax.axis_index("s") * NC + lax.axis_index("c")
    g0 = wid * GPW
    lanes = lax.iota(jnp.int32, L)

    uins = ((uin00, uin01), (uin10, uin11))
    iins = ((iin00, iin01), (iin10, iin11))
    uouts = (uout0, uout1)
    iouts = (iout0, iout1)

    def read(c, p):
        u0 = pl.multiple_of((g0 + c * GC) * 128, CU)
        cps = []
        for tr in range(2):
            cps.append(pltpu.async_copy(
                uwT_h.at[pl.ds(tr * 8, 8), pl.ds(u0, CU)], uins[p][tr], semR))
            cps.append(pltpu.async_copy(
                iwT_h.at[pl.ds(tr * 8, 8), pl.ds(u0, CU)], iins[p][tr], semR))
        return cps

    def transpose(p, n_users):
        for ins, out in ((uins[p], uouts[p]), (iins[p], iouts[p])):
            for j in range(n_users // L):
                bj = (j * L + lanes) * H
                for h in range(H):
                    v = ins[h // 8][h % 8, pl.ds(j * L, L)]
                    plsc.store_scatter(out, [bj + h], v)

    def write(c, p):
        u0 = pl.multiple_of((g0 + c * GC) * 128, CU)
        return [
            pltpu.async_copy(uouts[p], uscr_h.at[pl.ds(u0 * H, CU * H)], semW),
            pltpu.async_copy(iouts[p], iscr_h.at[pl.ds(u0 * H, CU * H)], semW),
        ]

    def pair(t, _):
        ca = 2 * t
        ra = read(ca, 0)
        rb = read(ca + 1, 1)
        for cp in ra:
            cp.wait()
        transpose(0, CU)
        wa = write(ca, 0)
        for cp in rb:
            cp.wait()
        transpose(1, CU)
        wb = write(ca + 1, 1)
        for cp in wa + wb:
            cp.wait()
        return 0

    lax.fori_loop(0, CPW // 2, pair, 0)

    def tail(nt_users):
        u0 = pl.multiple_of((NW * GPW + wid) * 128, 128)
        cps = []
        for tr in range(2):
            cps.append(pltpu.async_copy(
                uwT_h.at[pl.ds(tr * 8, 8), pl.ds(u0, nt_users)],
                uins[0][tr].at[:, pl.ds(0, nt_users)], semR))
            cps.append(pltpu.async_copy(
                iwT_h.at[pl.ds(tr * 8, 8), pl.ds(u0, nt_users)],
                iins[0][tr].at[:, pl.ds(0, nt_users)], semR))
        for cp in cps:
            cp.wait()
        transpose(0, nt_users)
        u0h = pl.multiple_of(u0 * H, 1024)
        wcps = [
            pltpu.async_copy(uouts[0].at[pl.ds(0, nt_users * H)],
                             uscr_h.at[pl.ds(u0h, nt_users * H)], semW),
            pltpu.async_copy(iouts[0].at[pl.ds(0, nt_users * H)],
                             iscr_h.at[pl.ds(u0h, nt_users * H)], semW),
        ]
        for cp in wcps:
            cp.wait()

    @pl.when(wid < NTAIL - 1)
    def _tail_full():
        tail(128)

    @pl.when(wid == NTAIL - 1)
    def _tail_part():
        tail(64)


@functools.partial(
    pl.kernel,
    out_type=(
        jax.ShapeDtypeStruct((B,), jnp.float32),
        jax.ShapeDtypeStruct((NW, L), jnp.float32),
    ),
    mesh=_mesh,
    compiler_params=_params,
    scratch_types=[
        pltpu.VMEM((BPW,), jnp.int32),
        pltpu.VMEM((BPW,), jnp.int32),
        pltpu.VMEM((NIDX,), jnp.int32),
        pltpu.VMEM((NIDX,), jnp.int32),
        pltpu.VMEM((NIDX,), jnp.float32),
        pltpu.VMEM((NIDX,), jnp.float32),
        pltpu.VMEM((BPW,), jnp.float32),
        pltpu.VMEM((BPW,), jnp.float32),
        pltpu.VMEM((BPW,), jnp.float32),
        pltpu.VMEM((BPW,), jnp.float32),
        pltpu.VMEM((L,), jnp.float32),
        pltpu.VMEM((L,), jnp.float32),
        pltpu.SemaphoreType.DMA,
        pltpu.SemaphoreType.DMA,
    ],
)
def _mf_kernel(user_h, item_h, rating_h, uscr_h, iscr_h, ub_h, ib_h, bias_h,
               tgt_h, part_h,
               uor_v, ior_v, uix_v, iix_v, ug_v, ig_v,
               ubr_v, ibr_v, rat_v, out_v, sqa_v, bias_v, semW, semB):
    wid = lax.axis_index("s") * NC + lax.axis_index("c")
    base = wid * BPW
    lanes = lax.iota(jnp.int32, L)

    pltpu.sync_copy(user_h.at[pl.ds(base, BPW)], uor_v)
    pltpu.sync_copy(item_h.at[pl.ds(base, BPW)], ior_v)
    pltpu.sync_copy(rating_h.at[pl.ds(base, BPW)], rat_v)
    pltpu.sync_copy(bias_h, bias_v)

    def build(j, _):
        bu = uor_v[pl.ds(j * L, L)] << 4
        bi = ior_v[pl.ds(j * L, L)] << 4
        for h in range(H):
            uix_v[pl.ds(h * BPW + j * L, L)] = bu + h
            iix_v[pl.ds(h * BPW + j * L, L)] = bi + h
        return 0

    lax.fori_loop(0, NB, build, 0)

    copies = []
    for k in range(NIDX // CH):
        sl = pl.ds(k * CH, CH)
        copies.append(pltpu.async_copy(uscr_h.at[uix_v.at[sl]], ug_v.at[sl], semW))
        copies.append(pltpu.async_copy(iscr_h.at[iix_v.at[sl]], ig_v.at[sl], semW))
    for q in range(BPW // CH):
        sl = pl.ds(q * CH, CH)
        copies.append(pltpu.async_copy(ub_h.at[uor_v.at[sl]], ubr_v.at[sl], semB))
        copies.append(pltpu.async_copy(ib_h.at[ior_v.at[sl]], ibr_v.at[sl], semB))
    for cp in copies:
        cp.wait()

    gbias = bias_v[...]

    def block(j, sqacc):
        o = j * L
        ub = ubr_v[pl.ds(o, L)]
        ib = ibr_v[pl.ds(o, L)]
        acc = jnp.zeros((L,), jnp.float32)
        for h in range(H):
            gu = ug_v[pl.ds(h * BPW + o, L)]
            gi = ig_v[pl.ds(h * BPW + o, L)]
            acc = acc + (gu + ub) * (gi + ib)
        out = acc + gbias
        out_v[pl.ds(o, L)] = out
        err = out - rat_v[pl.ds(o, L)]
        return sqacc + err * err

    sqacc = lax.fori_loop(0, NB, block, jnp.zeros((L,), jnp.float32))

    sqa_v[...] = sqacc
    pltpu.sync_copy(sqa_v, part_h.at[wid])
    pltpu.sync_copy(out_v, tgt_h.at[pl.ds(base, BPW)])


def kernel(user, item, rating, user_weight, item_weight, user_bias, item_bias,
           bias):
    user = user.astype(jnp.int32)
    item = item.astype(jnp.int32)
    bias16 = jnp.broadcast_to(bias.astype(jnp.float32), (L,))
    uscr, iscr = _relayout_kernel(user_weight.T, item_weight.T)
    target, parts = _mf_kernel(user, item, rating, uscr, iscr,
                               user_bias.reshape(U), item_bias.reshape(I),
                               bias16)
    loss = jnp.sum(parts) / B
    return (target, loss)


# speedup vs baseline: 3.4912x; 1.2136x over previous
import functools

import jax
import jax.numpy as jnp
from jax import lax
from jax.experimental import pallas as pl
from jax.experimental.pallas import tpu as pltpu
from jax.experimental.pallas import tpu_sc as plsc

B = 16384
U = 1000000
I = 1000000
H = 16
NC = 2
NS = 16
L = 16
NW = NC * NS
BPW = B // NW
NB = BPW // L
CH = 128
NIDX = BPW * H

NG = (U + 127) // 128
GPW = NG // NW
GC = 2
CPW = GPW // GC
CU = GC * 128
NTAIL = NG - GPW * NW

_mesh = plsc.VectorSubcoreMesh(core_axis_name="c", subcore_axis_name="s",
                               num_cores=NC, num_subcores=NS)
_params = pltpu.CompilerParams(needs_layout_passes=False,
                               use_tc_tiling_on_sc=True)


@functools.partial(
    pl.kernel,
    out_type=(
        jax.ShapeDtypeStruct((H * U,), jnp.float32),
        jax.ShapeDtypeStruct((H * I,), jnp.float32),
    ),
    mesh=_mesh,
    compiler_params=_params,
    scratch_types=[
        pltpu.VMEM((8, CU), jnp.float32),
        pltpu.VMEM((8, CU), jnp.float32),
        pltpu.VMEM((8, CU), jnp.float32),
        pltpu.VMEM((8, CU), jnp.float32),
        pltpu.VMEM((8, CU), jnp.float32),
        pltpu.VMEM((8, CU), jnp.float32),
        pltpu.VMEM((8, CU), jnp.float32),
        pltpu.VMEM((8, CU), jnp.float32),
        pltpu.VMEM((CU * H,), jnp.float32),
        pltpu.VMEM((CU * H,), jnp.float32),
        pltpu.VMEM((CU * H,), jnp.float32),
        pltpu.VMEM((CU * H,), jnp.float32),
        pltpu.SemaphoreType.DMA,
        pltpu.SemaphoreType.DMA,
        pltpu.SemaphoreType.DMA,
        pltpu.SemaphoreType.DMA,
    ],
)
def _relayout_kernel(uwT_h, iwT_h, uscr_h, iscr_h,
                     uin00, uin01, uin10, uin11,
                     iin00, iin01, iin10, iin11,
                     uout0, uout1, iout0, iout1, semR0, semR1, semW0, semW1):
    wid = lax.axis_index("s") * NC + lax.axis_index("c")
    g0 = wid * GPW
    lanes = lax.iota(jnp.int32, L)

    uins = ((uin00, uin01), (uin10, uin11))
    iins = ((iin00, iin01), (iin10, iin11))
    uouts = (uout0, uout1)
    iouts = (iout0, iout1)
    semRs = (semR0, semR1)
    semWs = (semW0, semW1)

    def read(c, p):
        u0 = pl.multiple_of((g0 + c * GC) * 128, CU)
        for tr in range(2):
            pltpu.async_copy(
                uwT_h.at[pl.ds(tr * 8, 8), pl.ds(u0, CU)], uins[p][tr], semRs[p])
            pltpu.async_copy(
                iwT_h.at[pl.ds(tr * 8, 8), pl.ds(u0, CU)], iins[p][tr], semRs[p])

    def drain_read(p):
        dummy = uwT_h.at[pl.ds(0, 8), pl.ds(0, CU)]
        for buf in (*uins[p], *iins[p]):
            pltpu.make_async_copy(dummy, buf, semRs[p]).wait()

    def drain_write(p):
        dummy = uscr_h.at[pl.ds(0, CU * H)]
        pltpu.make_async_copy(dummy, uouts[p], semWs[p]).wait()
        pltpu.make_async_copy(dummy, iouts[p], semWs[p]).wait()

    def transpose(p, n_users):
        for ins, out in ((uins[p], uouts[p]), (iins[p], iouts[p])):
            for j in range(n_users // L):
                bj = (j * L + lanes) * H
                for h in range(H):
                    v = ins[h // 8][h % 8, pl.ds(j * L, L)]
                    plsc.store_scatter(out, [bj + h], v)

    def write(c, p):
        u0 = pl.multiple_of((g0 + c * GC) * 128, CU)
        pltpu.async_copy(uouts[p], uscr_h.at[pl.ds(u0 * H, CU * H)], semWs[p])
        pltpu.async_copy(iouts[p], iscr_h.at[pl.ds(u0 * H, CU * H)], semWs[p])

    read(0, 0)
    read(1, 1)

    def pair(t, _):
        for p in range(2):
            c = 2 * t + p
            drain_read(p)

            @pl.when(t > 0)
            def _():
                drain_write(p)

            transpose(p, CU)
            write(c, p)

            @pl.when(t < CPW // 2 - 1)
            def _():
                read(c + 2, p)

        return 0

    lax.fori_loop(0, CPW // 2, pair, 0)
    drain_write(0)
    drain_write(1)

    def tail(nt_users):
        u0 = pl.multiple_of((NW * GPW + wid) * 128, 128)
        cps = []
        for tr in range(2):
            cps.append(pltpu.async_copy(
                uwT_h.at[pl.ds(tr * 8, 8), pl.ds(u0, nt_users)],
                uins[0][tr].at[:, pl.ds(0, nt_users)], semR0))
            cps.append(pltpu.async_copy(
                iwT_h.at[pl.ds(tr * 8, 8), pl.ds(u0, nt_users)],
                iins[0][tr].at[:, pl.ds(0, nt_users)], semR0))
        for cp in cps:
            cp.wait()
        transpose(0, nt_users)
        u0h = pl.multiple_of(u0 * H, 1024)
        wcps = [
            pltpu.async_copy(uouts[0].at[pl.ds(0, nt_users * H)],
                             uscr_h.at[pl.ds(u0h, nt_users * H)], semW0),
            pltpu.async_copy(iouts[0].at[pl.ds(0, nt_users * H)],
                             iscr_h.at[pl.ds(u0h, nt_users * H)], semW0),
        ]
        for cp in wcps:
            cp.wait()

    @pl.when(wid < NTAIL - 1)
    def _tail_full():
        tail(128)

    @pl.when(wid == NTAIL - 1)
    def _tail_part():
        tail(64)


@functools.partial(
    pl.kernel,
    out_type=(
        jax.ShapeDtypeStruct((B,), jnp.float32),
        jax.ShapeDtypeStruct((NW, L), jnp.float32),
    ),
    mesh=_mesh,
    compiler_params=_params,
    scratch_types=[
        pltpu.VMEM((BPW,), jnp.int32),
        pltpu.VMEM((BPW,), jnp.int32),
        pltpu.VMEM((NIDX,), jnp.int32),
        pltpu.VMEM((NIDX,), jnp.int32),
        pltpu.VMEM((NIDX,), jnp.float32),
        pltpu.VMEM((NIDX,), jnp.float32),
        pltpu.VMEM((BPW,), jnp.float32),
        pltpu.VMEM((BPW,), jnp.float32),
        pltpu.VMEM((BPW,), jnp.float32),
        pltpu.VMEM((BPW,), jnp.float32),
        pltpu.VMEM((L,), jnp.float32),
        pltpu.VMEM((L,), jnp.float32),
        pltpu.SemaphoreType.DMA,
        pltpu.SemaphoreType.DMA,
    ],
)
def _mf_kernel(user_h, item_h, rating_h, uscr_h, iscr_h, ub_h, ib_h, bias_h,
               tgt_h, part_h,
               uor_v, ior_v, uix_v, iix_v, ug_v, ig_v,
               ubr_v, ibr_v, rat_v, out_v, sqa_v, bias_v, semW, semB):
    wid = lax.axis_index("s") * NC + lax.axis_index("c")
    base = wid * BPW
    lanes = lax.iota(jnp.int32, L)

    pltpu.sync_copy(user_h.at[pl.ds(base, BPW)], uor_v)
    pltpu.sync_copy(item_h.at[pl.ds(base, BPW)], ior_v)
    pltpu.sync_copy(rating_h.at[pl.ds(base, BPW)], rat_v)
    pltpu.sync_copy(bias_h, bias_v)

    def build(j, _):
        bu = uor_v[pl.ds(j * L, L)] << 4
        bi = ior_v[pl.ds(j * L, L)] << 4
        for h in range(H):
            uix_v[pl.ds(h * BPW + j * L, L)] = bu + h
            iix_v[pl.ds(h * BPW + j * L, L)] = bi + h
        return 0

    lax.fori_loop(0, NB, build, 0)

    copies = []
    for k in range(NIDX // CH):
        sl = pl.ds(k * CH, CH)
        copies.append(pltpu.async_copy(uscr_h.at[uix_v.at[sl]], ug_v.at[sl], semW))
        copies.append(pltpu.async_copy(iscr_h.at[iix_v.at[sl]], ig_v.at[sl], semW))
    for q in range(BPW // CH):
        sl = pl.ds(q * CH, CH)
        copies.append(pltpu.async_copy(ub_h.at[uor_v.at[sl]], ubr_v.at[sl], semB))
        copies.append(pltpu.async_copy(ib_h.at[ior_v.at[sl]], ibr_v.at[sl], semB))
    for cp in copies:
        cp.wait()

    gbias = bias_v[...]

    def block(j, sqacc):
        o = j * L
        ub = ubr_v[pl.ds(o, L)]
        ib = ibr_v[pl.ds(o, L)]
        acc = jnp.zeros((L,), jnp.float32)
        for h in range(H):
            gu = ug_v[pl.ds(h * BPW + o, L)]
            gi = ig_v[pl.ds(h * BPW + o, L)]
            acc = acc + (gu + ub) * (gi + ib)
        out = acc + gbias
        out_v[pl.ds(o, L)] = out
        err = out - rat_v[pl.ds(o, L)]
        return sqacc + err * err

    sqacc = lax.fori_loop(0, NB, block, jnp.zeros((L,), jnp.float32))

    sqa_v[...] = sqacc
    pltpu.sync_copy(sqa_v, part_h.at[wid])
    pltpu.sync_copy(out_v, tgt_h.at[pl.ds(base, BPW)])


def kernel(user, item, rating, user_weight, item_weight, user_bias, item_bias,
           bias):
    user = user.astype(jnp.int32)
    item = item.astype(jnp.int32)
    bias16 = jnp.broadcast_to(bias.astype(jnp.float32), (L,))
    uscr, iscr = _relayout_kernel(user_weight.T, item_weight.T)
    target, parts = _mf_kernel(user, item, rating, uscr, iscr,
                               user_bias.reshape(U), item_bias.reshape(I),
                               bias16)
    loss = jnp.sum(parts) / B
    return (target, loss)


# speedup vs baseline: 8.0013x; 2.2919x over previous
import functools

import jax
import jax.numpy as jnp
from jax import lax
from jax.experimental import pallas as pl
from jax.experimental.pallas import tpu as pltpu
from jax.experimental.pallas import tpu_sc as plsc

B = 16384
U = 1000000
I = 1000000
H = 16
NC = 2
NS = 16
L = 16
NW = NC * NS
BPW = B // NW
NB = BPW // L
CH = 128
NIDX = BPW * H

NG = (U + 127) // 128
GPW = NG // NW
GC = 2
CPW = GPW // GC
CU = GC * 128
NTAIL = NG - GPW * NW

_mesh = plsc.VectorSubcoreMesh(core_axis_name="c", subcore_axis_name="s",
                               num_cores=NC, num_subcores=NS)
_params = pltpu.CompilerParams(needs_layout_passes=False,
                               use_tc_tiling_on_sc=True)


@functools.partial(
    pl.kernel,
    out_type=(
        jax.ShapeDtypeStruct((NG * 128 * H,), jnp.float32),
        jax.ShapeDtypeStruct((NG * 128 * H,), jnp.float32),
    ),
    mesh=_mesh,
    compiler_params=_params,
    scratch_types=[
        pltpu.VMEM((8, CU), jnp.float32),
        pltpu.VMEM((8, CU), jnp.float32),
        pltpu.VMEM((8, CU), jnp.float32),
        pltpu.VMEM((8, CU), jnp.float32),
        pltpu.VMEM((8, CU), jnp.float32),
        pltpu.VMEM((8, CU), jnp.float32),
        pltpu.VMEM((8, CU), jnp.float32),
        pltpu.VMEM((8, CU), jnp.float32),
        pltpu.VMEM((CU * H,), jnp.float32),
        pltpu.VMEM((CU * H,), jnp.float32),
        pltpu.VMEM((CU * H,), jnp.float32),
        pltpu.VMEM((CU * H,), jnp.float32),
        pltpu.SemaphoreType.DMA,
        pltpu.SemaphoreType.DMA,
        pltpu.SemaphoreType.DMA,
        pltpu.SemaphoreType.DMA,
    ],
)
def _relayout_kernel(uwT_h, iwT_h, uscr_h, iscr_h,
                     uin00, uin01, uin10, uin11,
                     iin00, iin01, iin10, iin11,
                     uout0, uout1, iout0, iout1, semR0, semR1, semW0, semW1):
    wid = lax.axis_index("s") * NC + lax.axis_index("c")
    g0 = wid * GPW
    lanes = lax.iota(jnp.int32, L)

    uins = ((uin00, uin01), (uin10, uin11))
    iins = ((iin00, iin01), (iin10, iin11))
    uouts = (uout0, uout1)
    iouts = (iout0, iout1)
    semRs = (semR0, semR1)
    semWs = (semW0, semW1)

    def read(c, p):
        u0 = pl.multiple_of((g0 + c * GC) * 128, CU)
        for tr in range(2):
            pltpu.async_copy(
                uwT_h.at[pl.ds(tr * 8, 8), pl.ds(u0, CU)], uins[p][tr], semRs[p])
            pltpu.async_copy(
                iwT_h.at[pl.ds(tr * 8, 8), pl.ds(u0, CU)], iins[p][tr], semRs[p])

    def drain_read(p):
        dummy = uwT_h.at[pl.ds(0, 8), pl.ds(0, CU)]
        for buf in (*uins[p], *iins[p]):
            pltpu.make_async_copy(dummy, buf, semRs[p]).wait()

    def drain_write(p):
        dummy = uscr_h.at[pl.ds(0, CU * H)]
        pltpu.make_async_copy(dummy, uouts[p], semWs[p]).wait()
        pltpu.make_async_copy(dummy, iouts[p], semWs[p]).wait()

    def transpose(p, n_users):
        for ins, out in ((uins[p], uouts[p]), (iins[p], iouts[p])):
            for g in range((n_users + 127) // 128):
                jn = min(8, (n_users - g * 128) // L)
                for h in range(H):
                    for j in range(jn):
                        out[pl.ds(g * 2048 + h * 128 + j * L, L)] = (
                            ins[h // 8][h % 8, pl.ds(g * 128 + j * L, L)])

    def write(c, p):
        u0 = pl.multiple_of((g0 + c * GC) * 128, CU)
        pltpu.async_copy(uouts[p], uscr_h.at[pl.ds(u0 * H, CU * H)], semWs[p])
        pltpu.async_copy(iouts[p], iscr_h.at[pl.ds(u0 * H, CU * H)], semWs[p])

    read(0, 0)
    read(1, 1)

    def pair(t, _):
        for p in range(2):
            c = 2 * t + p
            drain_read(p)

            @pl.when(t > 0)
            def _():
                drain_write(p)

            transpose(p, CU)
            write(c, p)

            @pl.when(t < CPW // 2 - 1)
            def _():
                read(c + 2, p)

        return 0

    lax.fori_loop(0, CPW // 2, pair, 0)
    drain_write(0)
    drain_write(1)

    def tail(nt_users):
        u0 = pl.multiple_of((NW * GPW + wid) * 128, 128)
        cps = []
        for tr in range(2):
            cps.append(pltpu.async_copy(
                uwT_h.at[pl.ds(tr * 8, 8), pl.ds(u0, nt_users)],
                uins[0][tr].at[:, pl.ds(0, nt_users)], semR0))
            cps.append(pltpu.async_copy(
                iwT_h.at[pl.ds(tr * 8, 8), pl.ds(u0, nt_users)],
                iins[0][tr].at[:, pl.ds(0, nt_users)], semR0))
        for cp in cps:
            cp.wait()
        transpose(0, nt_users)
        u0h = pl.multiple_of(u0 * H, 1024)
        wcps = [
            pltpu.async_copy(uouts[0].at[pl.ds(0, 128 * H)],
                             uscr_h.at[pl.ds(u0h, 128 * H)], semW0),
            pltpu.async_copy(iouts[0].at[pl.ds(0, 128 * H)],
                             iscr_h.at[pl.ds(u0h, 128 * H)], semW0),
        ]
        for cp in wcps:
            cp.wait()

    @pl.when(wid < NTAIL - 1)
    def _tail_full():
        tail(128)

    @pl.when(wid == NTAIL - 1)
    def _tail_part():
        tail(64)


@functools.partial(
    pl.kernel,
    out_type=(
        jax.ShapeDtypeStruct((B,), jnp.float32),
        jax.ShapeDtypeStruct((NW, L), jnp.float32),
    ),
    mesh=_mesh,
    compiler_params=_params,
    scratch_types=[
        pltpu.VMEM((BPW,), jnp.int32),
        pltpu.VMEM((BPW,), jnp.int32),
        pltpu.VMEM((NIDX,), jnp.int32),
        pltpu.VMEM((NIDX,), jnp.int32),
        pltpu.VMEM((NIDX,), jnp.float32),
        pltpu.VMEM((NIDX,), jnp.float32),
        pltpu.VMEM((BPW,), jnp.float32),
        pltpu.VMEM((BPW,), jnp.float32),
        pltpu.VMEM((BPW,), jnp.float32),
        pltpu.VMEM((BPW,), jnp.float32),
        pltpu.VMEM((L,), jnp.float32),
        pltpu.VMEM((L,), jnp.float32),
        pltpu.SemaphoreType.DMA,
        pltpu.SemaphoreType.DMA,
    ],
)
def _mf_kernel(user_h, item_h, rating_h, uscr_h, iscr_h, ub_h, ib_h, bias_h,
               tgt_h, part_h,
               uor_v, ior_v, uix_v, iix_v, ug_v, ig_v,
               ubr_v, ibr_v, rat_v, out_v, sqa_v, bias_v, semW, semB):
    wid = lax.axis_index("s") * NC + lax.axis_index("c")
    base = wid * BPW
    lanes = lax.iota(jnp.int32, L)

    pltpu.sync_copy(user_h.at[pl.ds(base, BPW)], uor_v)
    pltpu.sync_copy(item_h.at[pl.ds(base, BPW)], ior_v)
    pltpu.sync_copy(rating_h.at[pl.ds(base, BPW)], rat_v)
    pltpu.sync_copy(bias_h, bias_v)

    def build(j, _):
        ru = uor_v[pl.ds(j * L, L)]
        ri = ior_v[pl.ds(j * L, L)]
        bu = ((ru >> 7) << 11) + (ru & 127)
        bi = ((ri >> 7) << 11) + (ri & 127)
        for h in range(H):
            uix_v[pl.ds(h * BPW + j * L, L)] = bu + h * 128
            iix_v[pl.ds(h * BPW + j * L, L)] = bi + h * 128
        return 0

    lax.fori_loop(0, NB, build, 0)

    copies = []
    for k in range(NIDX // CH):
        sl = pl.ds(k * CH, CH)
        copies.append(pltpu.async_copy(uscr_h.at[uix_v.at[sl]], ug_v.at[sl], semW))
        copies.append(pltpu.async_copy(iscr_h.at[iix_v.at[sl]], ig_v.at[sl], semW))
    for q in range(BPW // CH):
        sl = pl.ds(q * CH, CH)
        copies.append(pltpu.async_copy(ub_h.at[uor_v.at[sl]], ubr_v.at[sl], semB))
        copies.append(pltpu.async_copy(ib_h.at[ior_v.at[sl]], ibr_v.at[sl], semB))
    for cp in copies:
        cp.wait()

    gbias = bias_v[...]

    def block(j, sqacc):
        o = j * L
        ub = ubr_v[pl.ds(o, L)]
        ib = ibr_v[pl.ds(o, L)]
        acc = jnp.zeros((L,), jnp.float32)
        for h in range(H):
            gu = ug_v[pl.ds(h * BPW + o, L)]
            gi = ig_v[pl.ds(h * BPW + o, L)]
            acc = acc + (gu + ub) * (gi + ib)
        out = acc + gbias
        out_v[pl.ds(o, L)] = out
        err = out - rat_v[pl.ds(o, L)]
        return sqacc + err * err

    sqacc = lax.fori_loop(0, NB, block, jnp.zeros((L,), jnp.float32))

    sqa_v[...] = sqacc
    pltpu.sync_copy(sqa_v, part_h.at[wid])
    pltpu.sync_copy(out_v, tgt_h.at[pl.ds(base, BPW)])


def kernel(user, item, rating, user_weight, item_weight, user_bias, item_bias,
           bias):
    user = user.astype(jnp.int32)
    item = item.astype(jnp.int32)
    bias16 = jnp.broadcast_to(bias.astype(jnp.float32), (L,))
    uscr, iscr = _relayout_kernel(user_weight.T, item_weight.T)
    target, parts = _mf_kernel(user, item, rating, uscr, iscr,
                               user_bias.reshape(U), item_bias.reshape(I),
                               bias16)
    loss = jnp.sum(parts) / B
    return (target, loss)
